# Initial kernel scaffold; baseline (speedup 1.0000x reference)
import functools

import jax
import jax.numpy as jnp
from jax import lax
from jax.experimental import pallas as pl
from jax.experimental.pallas import tpu as pltpu
from jax.experimental.pallas import tpu_sc as plsc

_CHUNK = 1024


@functools.lru_cache(maxsize=None)
def _make_gather(N, V, D):
    info = plsc.get_sparse_core_info()
    NC, NS = info.num_cores, info.num_subcores
    NW = NC * NS
    assert N % NW == 0
    b_per_w = N // NW
    chunk = min(_CHUNK, b_per_w)
    assert b_per_w % chunk == 0
    n_chunks = b_per_w // chunk
    mesh = plsc.VectorSubcoreMesh(core_axis_name="c", subcore_axis_name="s")

    @functools.partial(
        pl.kernel,
        mesh=mesh,
        out_type=jax.ShapeDtypeStruct((N, D), jnp.float32),
        scratch_types=[
            pltpu.VMEM((chunk,), jnp.int32),
            pltpu.VMEM((chunk, D), jnp.float32),
            pltpu.SemaphoreType.DMA,
        ],
    )
    def gather_kernel(table_hbm, idx_hbm, out_hbm, idx_v, rows_v, sem):
        wid = lax.axis_index("s") * NC + lax.axis_index("c")
        base = wid * b_per_w

        def body(i, carry):
            off = base + i * chunk
            pltpu.sync_copy(idx_hbm.at[pl.ds(off, chunk)], idx_v)
            pltpu.async_copy(table_hbm.at[idx_v], rows_v, sem).wait()
            pltpu.sync_copy(rows_v, out_hbm.at[pl.ds(off, chunk)])
            return carry

        lax.fori_loop(0, n_chunks, body, 0)

    return gather_kernel


def kernel(indices, table):
    B, H = indices.shape
    V, D = table.shape
    N = B * H
    idx_flat = indices.reshape(N).astype(jnp.int32)
    out = _make_gather(N, V, D)(table, idx_flat)
    return out.reshape(B, H, D)


# speedup vs baseline: 1.0941x; 1.0941x over previous
import functools

import jax
import jax.numpy as jnp
from jax import lax
from jax.experimental import pallas as pl
from jax.experimental.pallas import tpu as pltpu
from jax.experimental.pallas import tpu_sc as plsc

_CHUNK = 1024


@functools.lru_cache(maxsize=None)
def _make_gather(N, V, D):
    info = plsc.get_sparse_core_info()
    NC, NS = info.num_cores, info.num_subcores
    NW = NC * NS
    assert N % NW == 0
    b_per_w = N // NW
    chunk = min(_CHUNK, b_per_w)
    assert b_per_w % chunk == 0
    n_chunks = b_per_w // chunk
    mesh = plsc.VectorSubcoreMesh(core_axis_name="c", subcore_axis_name="s")

    @functools.partial(
        pl.kernel,
        mesh=mesh,
        out_type=jax.ShapeDtypeStruct((N, D), jnp.float32),
        scratch_types=[
            pltpu.VMEM((chunk,), jnp.int32),
            pltpu.VMEM((chunk, D), jnp.float32),
            pltpu.SemaphoreType.DMA,
        ],
        compiler_params=pltpu.CompilerParams(use_tc_tiling_on_sc=False),
    )
    def gather_kernel(table_hbm, idx_hbm, out_hbm, idx_v, rows_v, sem):
        wid = lax.axis_index("s") * NC + lax.axis_index("c")
        base = wid * b_per_w

        def body(i, carry):
            off = base + i * chunk
            pltpu.sync_copy(idx_hbm.at[pl.ds(off, chunk)], idx_v)
            pltpu.async_copy(table_hbm.at[idx_v], rows_v, sem).wait()
            pltpu.sync_copy(rows_v, out_hbm.at[pl.ds(off, chunk)])
            return carry

        lax.fori_loop(0, n_chunks, body, 0)

    return gather_kernel


def kernel(indices, table):
    B, H = indices.shape
    V, D = table.shape
    N = B * H
    idx_flat = indices.reshape(N).astype(jnp.int32)
    out = _make_gather(N, V, D)(table, idx_flat)
    return out.reshape(B, H, D)


# speedup vs baseline: 1.1083x; 1.0130x over previous
import functools

import jax
import jax.numpy as jnp
from jax import lax
from jax.experimental import pallas as pl
from jax.experimental.pallas import tpu as pltpu
from jax.experimental.pallas import tpu_sc as plsc

_CHUNK = 1600


@functools.lru_cache(maxsize=None)
def _make_gather(N, V, D):
    info = plsc.get_sparse_core_info()
    NC, NS = info.num_cores, info.num_subcores
    NW = NC * NS
    assert N % NW == 0
    b_per_w = N // NW
    chunk = min(_CHUNK, b_per_w)
    assert b_per_w % chunk == 0
    n_chunks = b_per_w // chunk
    mesh = plsc.VectorSubcoreMesh(core_axis_name="c", subcore_axis_name="s")

    @functools.partial(
        pl.kernel,
        mesh=mesh,
        out_type=jax.ShapeDtypeStruct((N, D), jnp.float32),
        scratch_types=[
            pltpu.VMEM((2, chunk), jnp.int32),
            pltpu.VMEM((2, chunk, D), jnp.float32),
            pltpu.SemaphoreType.DMA,
            pltpu.SemaphoreType.DMA,
            pltpu.SemaphoreType.DMA,
            pltpu.SemaphoreType.DMA,
        ],
        compiler_params=pltpu.CompilerParams(use_tc_tiling_on_sc=False),
    )
    def gather_kernel(table_hbm, idx_hbm, out_hbm, idx_v, rows_v,
                      gsem0, gsem1, ssem0, ssem1):
        wid = lax.axis_index("s") * NC + lax.axis_index("c")
        base = wid * b_per_w
        gsem = (gsem0, gsem1)
        ssem = (ssem0, ssem1)

        gathers = [None] * n_chunks
        stores = [None] * n_chunks
        for c in range(n_chunks):
            b = c & 1
            off = base + c * chunk
            if c >= 2:
                stores[c - 2].wait()
            pltpu.sync_copy(idx_hbm.at[pl.ds(off, chunk)], idx_v.at[b])
            gathers[c] = pltpu.async_copy(
                table_hbm.at[idx_v.at[b]], rows_v.at[b], gsem[b])
            if c >= 1:
                pb = (c - 1) & 1
                gathers[c - 1].wait()
                stores[c - 1] = pltpu.async_copy(
                    rows_v.at[pb],
                    out_hbm.at[pl.ds(base + (c - 1) * chunk, chunk)],
                    ssem[pb])
        last = n_chunks - 1
        b = last & 1
        gathers[last].wait()
        stores[last] = pltpu.async_copy(
            rows_v.at[b], out_hbm.at[pl.ds(base + last * chunk, chunk)],
            ssem[b])
        if n_chunks >= 2:
            stores[last - 1].wait()
        stores[last].wait()

    return gather_kernel


def kernel(indices, table):
    B, H = indices.shape
    V, D = table.shape
    N = B * H
    idx_flat = indices.reshape(N).astype(jnp.int32)
    out = _make_gather(N, V, D)(table, idx_flat)
    return out.reshape(B, H, D)


# speedup vs baseline: 1.1121x; 1.0034x over previous
import functools

import jax
import jax.numpy as jnp
from jax import lax
from jax.experimental import pallas as pl
from jax.experimental.pallas import tpu as pltpu
from jax.experimental.pallas import tpu_sc as plsc

_CHUNK = 640
_NBUF = 4


@functools.lru_cache(maxsize=None)
def _make_gather(N, V, D):
    info = plsc.get_sparse_core_info()
    NC, NS = info.num_cores, info.num_subcores
    NW = NC * NS
    assert N % NW == 0
    b_per_w = N // NW
    chunk = min(_CHUNK, b_per_w)
    nbuf = _NBUF
    group = nbuf * chunk
    assert b_per_w % group == 0
    n_groups = b_per_w // group
    mesh = plsc.VectorSubcoreMesh(core_axis_name="c", subcore_axis_name="s")

    @functools.partial(
        pl.kernel,
        mesh=mesh,
        out_type=jax.ShapeDtypeStruct((N, D), jnp.float32),
        scratch_types=[
            pltpu.VMEM((nbuf, chunk), jnp.int32),
            pltpu.VMEM((nbuf, chunk, D), jnp.float32),
        ] + [pltpu.SemaphoreType.DMA] * (2 * nbuf),
        compiler_params=pltpu.CompilerParams(use_tc_tiling_on_sc=False),
    )
    def gather_kernel(table_hbm, idx_hbm, out_hbm, idx_v, rows_v, *sems):
        gsem = sems[:nbuf]
        ssem = sems[nbuf:]
        wid = lax.axis_index("s") * NC + lax.axis_index("c")
        base = wid * b_per_w

        def body(g, carry):
            goff = base + g * group
            gathers = [None] * nbuf
            for b in range(nbuf):
                off = goff + b * chunk

                @pl.when(g >= 1)
                def _(b=b, off=off):
                    pltpu.make_async_copy(
                        rows_v.at[b],
                        out_hbm.at[pl.ds(off - group, chunk)],
                        ssem[b]).wait()

                pltpu.sync_copy(idx_hbm.at[pl.ds(off, chunk)], idx_v.at[b])
                gathers[b] = pltpu.async_copy(
                    table_hbm.at[idx_v.at[b]], rows_v.at[b], gsem[b])
            for b in range(nbuf):
                off = goff + b * chunk
                gathers[b].wait()
                pltpu.async_copy(
                    rows_v.at[b], out_hbm.at[pl.ds(off, chunk)], ssem[b])
            return carry

        lax.fori_loop(0, n_groups, body, 0)
        for b in range(nbuf):
            off = base + (n_groups - 1) * group + b * chunk
            pltpu.make_async_copy(
                rows_v.at[b], out_hbm.at[pl.ds(off, chunk)], ssem[b]).wait()

    return gather_kernel


def kernel(indices, table):
    B, H = indices.shape
    V, D = table.shape
    N = B * H
    idx_flat = indices.reshape(N).astype(jnp.int32)
    out = _make_gather(N, V, D)(table, idx_flat)
    return out.reshape(B, H, D)


# speedup vs baseline: 1.2626x; 1.1354x over previous
import functools

import jax
import jax.numpy as jnp
from jax import lax
from jax.experimental import pallas as pl
from jax.experimental.pallas import tpu as pltpu
from jax.experimental.pallas import tpu_sc as plsc

_BT = 128


@functools.lru_cache(maxsize=None)
def _make_gather(B, H, V, D):
    info = plsc.get_sparse_core_info()
    NC, NS, L = info.num_cores, info.num_subcores, info.num_lanes
    NW = NC * NS
    DT = D // 8
    n_bt = B // _BT
    bt_per_w = n_bt // NW
    blk = _BT * H
    mesh = plsc.VectorSubcoreMesh(core_axis_name="c", subcore_axis_name="s")

    @functools.partial(
        pl.kernel,
        mesh=mesh,
        out_type=jax.ShapeDtypeStruct((H, DT, n_bt, 8, L * 8), jnp.float32),
        scratch_types=[
            pltpu.VMEM((blk,), jnp.int32),
            pltpu.VMEM((_BT,), jnp.int32),
            pltpu.VMEM((_BT, D), jnp.float32),
            pltpu.VMEM((DT, 8, _BT), jnp.float32),
            pltpu.SemaphoreType.DMA,
        ],
        compiler_params=pltpu.CompilerParams(
            use_tc_tiling_on_sc=False, needs_layout_passes=False),
    )
    def gather_kernel(table_hbm, idx_hbm, out_hbm, idxblk, glist, rows,
                      slab, gsem):
        wid = lax.axis_index("s") * NC + lax.axis_index("c")
        iota = lax.iota(jnp.int32, L)
        bl_vecs = [j * L + iota for j in range(_BT // L)]
        d_vecs = [jnp.full((L,), d, jnp.int32) for d in range(D)]

        for t in range(bt_per_w):
            bt = wid * bt_per_w + t
            pltpu.sync_copy(idx_hbm.at[pl.ds(bt * blk, blk)], idxblk)

            def h_body(h, carry, bt=bt):
                for j in range(_BT // L):
                    pos = bl_vecs[j] * H + h
                    glist[pl.ds(j * L, L)] = plsc.load_gather(idxblk, [pos])
                pltpu.async_copy(table_hbm.at[glist], rows, gsem).wait()
                for dt in range(DT):
                    for ds in range(8):
                        d = dt * 8 + ds
                        for j in range(_BT // L):
                            slab[dt, ds, pl.ds(j * L, L)] = plsc.load_gather(
                                rows, [bl_vecs[j], d_vecs[d]])
                pltpu.sync_copy(slab, out_hbm.at[h, :, bt, :, :])
                return carry

            lax.fori_loop(0, H, h_body, 0)

    return gather_kernel


def kernel(indices, table):
    B, H = indices.shape
    V, D = table.shape
    idx_flat = indices.reshape(B * H).astype(jnp.int32)
    o5 = _make_gather(B, H, V, D)(table, idx_flat)
    return o5.transpose(2, 4, 0, 1, 3).reshape(B, H, D)


# speedup vs baseline: 1.6043x; 1.2706x over previous
import functools

import jax
import jax.numpy as jnp
from jax import lax
from jax.experimental import pallas as pl
from jax.experimental.pallas import tpu as pltpu
from jax.experimental.pallas import tpu_sc as plsc

_BT = 128
_QB = 32


@functools.lru_cache(maxsize=None)
def _make_gather(B, H, V, D):
    info = plsc.get_sparse_core_info()
    NC, NS, L = info.num_cores, info.num_subcores, info.num_lanes
    NW = NC * NS
    DT = D // 8
    n_bt = B // _BT
    uq = _BT // _QB
    urows = _QB * H
    n_units = n_bt * uq
    upw = n_units // NW
    assert H % 2 == 0
    mesh = plsc.VectorSubcoreMesh(core_axis_name="c", subcore_axis_name="s")

    @functools.partial(
        pl.kernel,
        mesh=mesh,
        out_type=jax.ShapeDtypeStruct((H, DT, n_bt, 8, _BT), jnp.float32),
        scratch_types=[
            pltpu.VMEM((2, urows), jnp.int32),
            pltpu.VMEM((2, urows, D), jnp.float32),
            pltpu.VMEM((DT, 8, _QB), jnp.float32),
            pltpu.VMEM((DT, 8, _QB), jnp.float32),
            pltpu.SemaphoreType.DMA,
            pltpu.SemaphoreType.DMA,
            pltpu.SemaphoreType.DMA,
            pltpu.SemaphoreType.DMA,
        ],
        compiler_params=pltpu.CompilerParams(
            use_tc_tiling_on_sc=False, needs_layout_passes=False),
    )
    def gather_kernel(table_hbm, idx_hbm, out_hbm, idxq, rows, slab_a,
                      slab_b, gsem0, gsem1, ssem_a, ssem_b):
        wid = lax.axis_index("s") * NC + lax.axis_index("c")
        u0 = wid * upw
        iota = lax.iota(jnp.int32, L)
        jvec = [(j * L + iota) * H for j in range(_QB // L)]
        dvec = [jnp.full((L,), d, jnp.int32) for d in range(D)]
        slabs = (slab_a, slab_b)
        ssems = (ssem_a, ssem_b)
        gsems = (gsem0, gsem1)

        def start_gather(u, b):
            pltpu.sync_copy(
                idx_hbm.at[pl.ds((u0 + u) * urows, urows)], idxq.at[b])
            return pltpu.async_copy(table_hbm.at[idxq.at[b]], rows.at[b],
                                    gsems[b])

        def transpose_unit(u, b):
            bt = (u0 + u) // uq
            bl0 = ((u0 + u) % uq) * _QB
            rbuf = rows.at[b]

            def p_body(p, carry):
                for s in range(2):
                    h = 2 * p + s

                    @pl.when(p >= 1)
                    def _(s=s, h=h):
                        pltpu.make_async_copy(
                            slabs[s],
                            out_hbm.at[h, :, bt, :, pl.ds(bl0, _QB)],
                            ssems[s]).wait()

                    lvec = [jv + h for jv in jvec]
                    for dt in range(DT):
                        for ds in range(8):
                            d = dt * 8 + ds
                            for j in range(_QB // L):
                                slabs[s][dt, ds, pl.ds(j * L, L)] = (
                                    plsc.load_gather(
                                        rbuf, [lvec[j], dvec[d]]))
                    pltpu.async_copy(
                        slabs[s],
                        out_hbm.at[h, :, bt, :, pl.ds(bl0, _QB)],
                        ssems[s])
                return carry

            lax.fori_loop(0, H // 2, p_body, 0)
            for s in range(2):
                pltpu.make_async_copy(
                    slabs[s],
                    out_hbm.at[H - 2 + s, :, bt, :, pl.ds(bl0, _QB)],
                    ssems[s]).wait()

        gather = start_gather(0, 0)
        for u in range(upw):
            b = u & 1
            nxt = None
            if u + 1 < upw:
                nxt = start_gather(u + 1, 1 - b)
            gather.wait()
            transpose_unit(u, b)
            gather = nxt

    return gather_kernel


def kernel(indices, table):
    B, H = indices.shape
    V, D = table.shape
    idx_flat = indices.reshape(B * H).astype(jnp.int32)
    o5 = _make_gather(B, H, V, D)(table, idx_flat)
    return o5.transpose(2, 4, 0, 1, 3).reshape(B, H, D)


# speedup vs baseline: 1.9417x; 1.2103x over previous
import functools

import jax
import jax.numpy as jnp
from jax import lax
from jax.experimental import pallas as pl
from jax.experimental.pallas import tpu as pltpu
from jax.experimental.pallas import tpu_sc as plsc

_BT = 128
_QB = 32


@functools.lru_cache(maxsize=None)
def _make_gather(B, H, V, D):
    info = plsc.get_sparse_core_info()
    NC, NS, L = info.num_cores, info.num_subcores, info.num_lanes
    NW = NC * NS
    DT = D // 8
    n_bt = B // _BT
    uq = _BT // _QB
    urows = _QB * H
    n_units = n_bt * uq
    upw = n_units // NW
    assert H % 2 == 0
    mesh = plsc.VectorSubcoreMesh(core_axis_name="c", subcore_axis_name="s")

    @functools.partial(
        pl.kernel,
        mesh=mesh,
        out_type=jax.ShapeDtypeStruct((H, DT, n_bt, 8, _BT), jnp.float32),
        scratch_types=[
            pltpu.VMEM((2, urows), jnp.int32),
            pltpu.VMEM((2, urows, D), jnp.float32),
            pltpu.VMEM((DT, 8, _QB), jnp.float32),
            pltpu.VMEM((DT, 8, _QB), jnp.float32),
            pltpu.SemaphoreType.DMA,
            pltpu.SemaphoreType.DMA,
            pltpu.SemaphoreType.DMA,
            pltpu.SemaphoreType.DMA,
        ],
        compiler_params=pltpu.CompilerParams(
            use_tc_tiling_on_sc=False, needs_layout_passes=False),
    )
    def gather_kernel(table_hbm, idx_hbm, out_hbm, idxq, rows, slab_a,
                      slab_b, gsem0, gsem1, ssem_a, ssem_b):
        wid = lax.axis_index("s") * NC + lax.axis_index("c")
        u0 = wid * upw
        iota = lax.iota(jnp.int32, L)
        jvec = [(j * L + iota) * H for j in range(_QB // L)]
        dvec = [jnp.full((L,), d, jnp.int32) for d in range(D)]
        slabs = (slab_a, slab_b)
        ssems = (ssem_a, ssem_b)
        gsems = (gsem0, gsem1)

        def start_gather(u, b):
            pltpu.sync_copy(
                idx_hbm.at[pl.ds((u0 + u) * urows, urows)], idxq.at[b])
            return pltpu.async_copy(table_hbm.at[idxq.at[b]], rows.at[b],
                                    gsems[b])

        def transpose_unit(u, b):
            bt = (u0 + u) // uq
            bl0 = ((u0 + u) % uq) * _QB
            rbuf = rows.at[b]

            def p_body(p, carry):
                for s in range(2):
                    h = 2 * p + s

                    @pl.when(p >= 1)
                    def _(s=s, h=h):
                        pltpu.make_async_copy(
                            slabs[s],
                            out_hbm.at[h, :, bt, :, pl.ds(bl0, _QB)],
                            ssems[s]).wait()

                    lvec = [jv + h for jv in jvec]
                    nj = _QB // L
                    for dt in range(DT):
                        vs = [
                            plsc.load_gather(
                                rbuf, [lvec[j], dvec[dt * 8 + ds]])
                            for ds in range(8)
                            for j in range(nj)
                        ]
                        for ds in range(8):
                            for j in range(nj):
                                slabs[s][dt, ds, pl.ds(j * L, L)] = (
                                    vs[ds * nj + j])
                    pltpu.async_copy(
                        slabs[s],
                        out_hbm.at[h, :, bt, :, pl.ds(bl0, _QB)],
                        ssems[s])
                return carry

            lax.fori_loop(0, H // 2, p_body, 0)
            for s in range(2):
                pltpu.make_async_copy(
                    slabs[s],
                    out_hbm.at[H - 2 + s, :, bt, :, pl.ds(bl0, _QB)],
                    ssems[s]).wait()

        gather = start_gather(0, 0)
        for u in range(upw):
            b = u & 1
            nxt = None
            if u + 1 < upw:
                nxt = start_gather(u + 1, 1 - b)
            gather.wait()
            transpose_unit(u, b)
            gather = nxt

    return gather_kernel


def kernel(indices, table):
    B, H = indices.shape
    V, D = table.shape
    idx_flat = indices.reshape(B * H).astype(jnp.int32)
    o5 = _make_gather(B, H, V, D)(table, idx_flat)
    return o5.transpose(2, 4, 0, 1, 3).reshape(B, H, D)
